# baseline (speedup 1.0000x reference)
import functools

import numpy as np
import jax
import jax.numpy as jnp
from jax import lax
from jax.experimental import pallas as pl
from jax.experimental.pallas import tpu as pltpu
from jax.experimental.pallas import tpu_sc as plsc

NC, NS, L = 2, 16, 16
NW = NC * NS
BINS = 64
NIMG = 16
NCH = NIMG * 3
W = 512
ROWS_PER_CH = 512
TASKS_PER_CH = 2
NTASKS = NCH * TASKS_PER_CH
ROWS_PER_TASK = ROWS_PER_CH // TASKS_PER_CH
TASKS_PER_W = NTASKS // NW
ROWS = 64
CHUNKS_PER_TASK = ROWS_PER_TASK // ROWS
VPR = W // 16
NPH = 16
PHW = BINS * L
TASK_WORDS = PHW


def _sc_hist_body(img_hbm, out_hbm, buf0, buf1, acc, sem0, sem1):
    c = lax.axis_index("c")
    s = lax.axis_index("s")
    w = c * NS + s
    lane = lax.iota(jnp.int32, 16)
    lane_u = [lane + u * PHW for u in range(NPH)]
    zero = jnp.zeros((16,), jnp.float32)
    one = jnp.ones((16,), jnp.float32)

    bufs = (buf0, buf1)
    sems = (sem0, sem1)

    def task_body(ti, carry):
        t = w + NW * ti
        ch = t // TASKS_PER_CH
        rbase = (t % TASKS_PER_CH) * ROWS_PER_TASK

        def start(ci):
            cp = pltpu.make_async_copy(
                img_hbm.at[ch, pl.ds(rbase + ci * ROWS, ROWS), :],
                bufs[ci % 2], sems[ci % 2])
            cp.start()
            return cp

        pending = [None, None]
        pending[0] = start(0)

        @plsc.parallel_loop(0, NPH * PHW // 16, unroll=4)
        def _(i):
            acc[pl.ds(i * 16, 16)] = zero

        for ci in range(CHUNKS_PER_TASK):
            if ci + 1 < CHUNKS_PER_TASK:
                pending[(ci + 1) % 2] = start(ci + 1)
            pending[ci % 2].wait()
            buf = bufs[ci % 2]

            @plsc.parallel_loop(0, ROWS, step=1, unroll=1)
            def _(r, buf=buf):
                for u in range(VPR):
                    x = buf[r, pl.ds(u * 16, 16)]
                    b = (x * 64.0).astype(jnp.int32)
                    plsc.addupdate_scatter(
                        acc, [lane_u[u % NPH] + (b << 4)], one)

        @plsc.parallel_loop(0, PHW // 16, unroll=4)
        def _(i):
            tot = acc[pl.ds(i * 16, 16)]
            for u in range(1, NPH):
                tot = tot + acc[pl.ds(u * PHW + i * 16, 16)]
            acc[pl.ds(i * 16, 16)] = tot

        pltpu.sync_copy(acc.at[pl.ds(0, TASK_WORDS)], out_hbm.at[t])
        return carry

    lax.fori_loop(0, TASKS_PER_W, task_body, 0)


_sc_hist = functools.partial(
    pl.kernel,
    out_type=jax.ShapeDtypeStruct((NTASKS, TASK_WORDS), jnp.float32),
    mesh=plsc.VectorSubcoreMesh(
        core_axis_name="c", subcore_axis_name="s",
        num_cores=NC, num_subcores=NS),
    scratch_types=[
        pltpu.VMEM((ROWS, W), jnp.float32),
        pltpu.VMEM((ROWS, W), jnp.float32),
        pltpu.VMEM((NPH * PHW,), jnp.float32),
        pltpu.SemaphoreType.DMA,
        pltpu.SemaphoreType.DMA,
    ],
    compiler_params=pltpu.CompilerParams(
        needs_layout_passes=False, use_tc_tiling_on_sc=True),
)(_sc_hist_body)

_J = np.arange(3 * TASKS_PER_CH * PHW)
_EXPAND = np.zeros((3 * TASKS_PER_CH * PHW, 3 * BINS), np.float32)
_EXPAND[_J, (_J // (TASKS_PER_CH * PHW)) * BINS + (_J % PHW) // L] = 1.0


def _mlp_body(p_ref, m_ref, w1, b1, w2, b2, w3, b3, g, o_ref):
    q = p_ref[...]
    hist = jnp.dot(q, m_ref[...], preferred_element_type=jnp.float32)
    f = jnp.dot(hist, w1[...], preferred_element_type=jnp.float32) + b1[...]
    f = jnp.maximum(f, 0.0)
    f = jnp.dot(f, w2[...], preferred_element_type=jnp.float32) + b2[...]
    f = jnp.maximum(f, 0.0)
    f = jnp.dot(f, w3[...], preferred_element_type=jnp.float32) + b3[...]
    o_ref[...] = jax.nn.sigmoid(g[0, 0] + f)


_mlp = pl.pallas_call(
    _mlp_body,
    out_shape=jax.ShapeDtypeStruct((NIMG, BINS), jnp.float32),
)


def kernel(img, params):
    img3 = img.reshape(NCH, ROWS_PER_CH, W)
    partial = _sc_hist(img3)
    p2 = partial.reshape(NIMG, 3 * TASKS_PER_CH * PHW)
    w1 = params[0:49152].reshape(192, 256)
    b1 = params[49152:49408].reshape(1, 256)
    w2 = params[49408:82176].reshape(256, 128)
    b2 = params[82176:82304].reshape(1, 128)
    w3 = params[82304:90496].reshape(128, 64)
    b3 = params[90496:90560].reshape(1, 64)
    g = params[90560:90561].reshape(1, 1)
    return _mlp(p2, jnp.asarray(_EXPAND), w1, b1, w2, b2, w3, b3, g)


# speedup vs baseline: 1.0387x; 1.0387x over previous
import functools

import numpy as np
import jax
import jax.numpy as jnp
from jax import lax
from jax.experimental import pallas as pl
from jax.experimental.pallas import tpu as pltpu
from jax.experimental.pallas import tpu_sc as plsc

NC, NS, L = 2, 16, 16
NW = NC * NS
BINS = 64
NIMG = 16
NCH = NIMG * 3
W = 512
ROWS_PER_CH = 512
TASKS_PER_CH = 2
NTASKS = NCH * TASKS_PER_CH
ROWS_PER_TASK = ROWS_PER_CH // TASKS_PER_CH
TASKS_PER_W = NTASKS // NW
ROWS = 64
CHUNKS_PER_TASK = ROWS_PER_TASK // ROWS
VPR = W // 16
NPH = 16
PHW = BINS * L
TASK_WORDS = PHW


def _sc_hist_body(img_hbm, out_hbm, buf0, buf1, acc, sem0, sem1):
    c = lax.axis_index("c")
    s = lax.axis_index("s")
    w = c * NS + s
    lane = lax.iota(jnp.int32, 16)
    lane_u = [lane + u * PHW for u in range(NPH)]
    zero = jnp.zeros((16,), jnp.float32)
    one = jnp.ones((16,), jnp.float32)

    bufs = (buf0, buf1)
    sems = (sem0, sem1)
    nchunks = TASKS_PER_W * CHUNKS_PER_TASK

    def start(k):
        ti, ci = divmod(k, CHUNKS_PER_TASK)
        t = w + NW * ti
        ch = t // TASKS_PER_CH
        row0 = (t % TASKS_PER_CH) * ROWS_PER_TASK + ci * ROWS
        cp = pltpu.make_async_copy(
            img_hbm.at[ch, pl.ds(row0, ROWS), :], bufs[k % 2], sems[k % 2])
        cp.start()
        return cp

    pending = [None, None]
    pending[0] = start(0)
    for k in range(nchunks):
        if k + 1 < nchunks:
            pending[(k + 1) % 2] = start(k + 1)

        if k % CHUNKS_PER_TASK == 0:
            @plsc.parallel_loop(0, NPH * PHW // 16, unroll=4)
            def _(i):
                acc[pl.ds(i * 16, 16)] = zero

        pending[k % 2].wait()
        buf = bufs[k % 2]

        @plsc.parallel_loop(0, ROWS * 2, step=1, unroll=2)
        def _(h, buf=buf):
            r = h >> 1
            c0 = (h & 1) * (W // 2)
            for u in range(VPR // 2):
                x = buf[r, pl.ds(c0 + u * 16, 16)]
                b = (x * 64.0).astype(jnp.int32)
                plsc.addupdate_scatter(acc, [lane_u[u] + (b << 4)], one)

        if k % CHUNKS_PER_TASK == CHUNKS_PER_TASK - 1:
            @plsc.parallel_loop(0, PHW // 16, unroll=4)
            def _(i):
                tot = acc[pl.ds(i * 16, 16)]
                for u in range(1, NPH):
                    tot = tot + acc[pl.ds(u * PHW + i * 16, 16)]
                acc[pl.ds(i * 16, 16)] = tot

            ti = k // CHUNKS_PER_TASK
            t = w + NW * ti
            pltpu.sync_copy(acc.at[pl.ds(0, TASK_WORDS)], out_hbm.at[t])


_sc_hist = functools.partial(
    pl.kernel,
    out_type=jax.ShapeDtypeStruct((NTASKS, TASK_WORDS), jnp.float32),
    mesh=plsc.VectorSubcoreMesh(
        core_axis_name="c", subcore_axis_name="s",
        num_cores=NC, num_subcores=NS),
    scratch_types=[
        pltpu.VMEM((ROWS, W), jnp.float32),
        pltpu.VMEM((ROWS, W), jnp.float32),
        pltpu.VMEM((NPH * PHW,), jnp.float32),
        pltpu.SemaphoreType.DMA,
        pltpu.SemaphoreType.DMA,
    ],
    compiler_params=pltpu.CompilerParams(
        needs_layout_passes=False, use_tc_tiling_on_sc=True),
)(_sc_hist_body)

_J = np.arange(3 * TASKS_PER_CH * PHW)
_EXPAND = np.zeros((3 * TASKS_PER_CH * PHW, 3 * BINS), np.float32)
_EXPAND[_J, (_J // (TASKS_PER_CH * PHW)) * BINS + (_J % PHW) // L] = 1.0


def _mlp_body(p_ref, m_ref, w1, b1, w2, b2, w3, b3, g, o_ref):
    q = p_ref[...]
    hist = jnp.dot(q, m_ref[...], preferred_element_type=jnp.float32)
    f = jnp.dot(hist, w1[...], preferred_element_type=jnp.float32) + b1[...]
    f = jnp.maximum(f, 0.0)
    f = jnp.dot(f, w2[...], preferred_element_type=jnp.float32) + b2[...]
    f = jnp.maximum(f, 0.0)
    f = jnp.dot(f, w3[...], preferred_element_type=jnp.float32) + b3[...]
    o_ref[...] = jax.nn.sigmoid(g[0, 0] + f)


_mlp = pl.pallas_call(
    _mlp_body,
    out_shape=jax.ShapeDtypeStruct((NIMG, BINS), jnp.float32),
)


def kernel(img, params):
    img3 = img.reshape(NCH, ROWS_PER_CH, W)
    partial = _sc_hist(img3)
    p2 = partial.reshape(NIMG, 3 * TASKS_PER_CH * PHW)
    w1 = params[0:49152].reshape(192, 256)
    b1 = params[49152:49408].reshape(1, 256)
    w2 = params[49408:82176].reshape(256, 128)
    b2 = params[82176:82304].reshape(1, 128)
    w3 = params[82304:90496].reshape(128, 64)
    b3 = params[90496:90560].reshape(1, 64)
    g = params[90560:90561].reshape(1, 1)
    return _mlp(p2, jnp.asarray(_EXPAND), w1, b1, w2, b2, w3, b3, g)


# speedup vs baseline: 1.1923x; 1.1479x over previous
import functools

import numpy as np
import jax
import jax.numpy as jnp
from jax import lax
from jax.experimental import pallas as pl
from jax.experimental.pallas import tpu as pltpu
from jax.experimental.pallas import tpu_sc as plsc

NC, NS, L = 2, 16, 16
NW = NC * NS
BINS = 64
NIMG = 16
NCH = NIMG * 3
W = 512
ROWS_PER_CH = 512
TASKS_PER_CH = 2
NTASKS = NCH * TASKS_PER_CH
ROWS_PER_TASK = ROWS_PER_CH // TASKS_PER_CH
TASKS_PER_W = NTASKS // NW
ROWS = 64
CHUNKS_PER_TASK = ROWS_PER_TASK // ROWS
VPR = W // 16
NPH = 16
PHW = BINS * L
TASK_WORDS = PHW


def _sc_hist_body(img_hbm, out_hbm, buf0, buf1, acc, sem0, sem1):
    c = lax.axis_index("c")
    s = lax.axis_index("s")
    w = c * NS + s
    lane = lax.iota(jnp.int32, 16)
    lane_u = [lane + u * PHW for u in range(NPH)]
    zero = jnp.zeros((16,), jnp.float32)
    one = jnp.ones((16,), jnp.float32)

    bufs = (buf0, buf1)
    sems = (sem0, sem1)
    nchunks = TASKS_PER_W * CHUNKS_PER_TASK

    def start(k):
        ti, ci = divmod(k, CHUNKS_PER_TASK)
        t = w + NW * ti
        ch = t // TASKS_PER_CH
        row0 = (t % TASKS_PER_CH) * ROWS_PER_TASK + ci * ROWS
        cp = pltpu.make_async_copy(
            img_hbm.at[ch, pl.ds(row0, ROWS), :], bufs[k % 2], sems[k % 2])
        cp.start()
        return cp

    pending = [None, None]
    pending[0] = start(0)
    for k in range(nchunks):
        if k + 1 < nchunks:
            pending[(k + 1) % 2] = start(k + 1)

        if k % CHUNKS_PER_TASK == 0:
            @plsc.parallel_loop(0, NPH * PHW // 16, unroll=4)
            def _(i):
                acc[pl.ds(i * 16, 16)] = zero

        pending[k % 2].wait()
        buf = bufs[k % 2]

        @plsc.parallel_loop(0, ROWS * 2, step=1, unroll=2)
        def _(h, buf=buf):
            r = h >> 1
            c0 = (h & 1) * (W // 2)
            for u in range(VPR // 2):
                x = buf[r, pl.ds(c0 + u * 16, 16)]
                y = jnp.maximum(x * 64.0 + 524287.96875, 524288.0)
                b16 = plsc.bitcast(y, jnp.int32) & 0x3F0
                plsc.addupdate_scatter(acc, [lane_u[u] | b16], one)

        if k % CHUNKS_PER_TASK == CHUNKS_PER_TASK - 1:
            @plsc.parallel_loop(0, PHW // 16, unroll=4)
            def _(i):
                tot = acc[pl.ds(i * 16, 16)]
                for u in range(1, NPH):
                    tot = tot + acc[pl.ds(u * PHW + i * 16, 16)]
                acc[pl.ds(i * 16, 16)] = tot

            ti = k // CHUNKS_PER_TASK
            t = w + NW * ti
            pltpu.sync_copy(acc.at[pl.ds(0, TASK_WORDS)], out_hbm.at[t])


_sc_hist = functools.partial(
    pl.kernel,
    out_type=jax.ShapeDtypeStruct((NTASKS, TASK_WORDS), jnp.float32),
    mesh=plsc.VectorSubcoreMesh(
        core_axis_name="c", subcore_axis_name="s",
        num_cores=NC, num_subcores=NS),
    scratch_types=[
        pltpu.VMEM((ROWS, W), jnp.float32),
        pltpu.VMEM((ROWS, W), jnp.float32),
        pltpu.VMEM((NPH * PHW,), jnp.float32),
        pltpu.SemaphoreType.DMA,
        pltpu.SemaphoreType.DMA,
    ],
    compiler_params=pltpu.CompilerParams(
        needs_layout_passes=False, use_tc_tiling_on_sc=True),
)(_sc_hist_body)

_J = np.arange(3 * TASKS_PER_CH * PHW)
_EXPAND = np.zeros((3 * TASKS_PER_CH * PHW, 3 * BINS), np.float32)
_EXPAND[_J, (_J // (TASKS_PER_CH * PHW)) * BINS + (_J % PHW) // L] = 1.0


def _mlp_body(p_ref, m_ref, w1, b1, w2, b2, w3, b3, g, o_ref):
    q = p_ref[...]
    hist = jnp.dot(q, m_ref[...], preferred_element_type=jnp.float32)
    f = jnp.dot(hist, w1[...], preferred_element_type=jnp.float32) + b1[...]
    f = jnp.maximum(f, 0.0)
    f = jnp.dot(f, w2[...], preferred_element_type=jnp.float32) + b2[...]
    f = jnp.maximum(f, 0.0)
    f = jnp.dot(f, w3[...], preferred_element_type=jnp.float32) + b3[...]
    o_ref[...] = jax.nn.sigmoid(g[0, 0] + f)


_mlp = pl.pallas_call(
    _mlp_body,
    out_shape=jax.ShapeDtypeStruct((NIMG, BINS), jnp.float32),
)


def kernel(img, params):
    img3 = img.reshape(NCH, ROWS_PER_CH, W)
    partial = _sc_hist(img3)
    p2 = partial.reshape(NIMG, 3 * TASKS_PER_CH * PHW)
    w1 = params[0:49152].reshape(192, 256)
    b1 = params[49152:49408].reshape(1, 256)
    w2 = params[49408:82176].reshape(256, 128)
    b2 = params[82176:82304].reshape(1, 128)
    w3 = params[82304:90496].reshape(128, 64)
    b3 = params[90496:90560].reshape(1, 64)
    g = params[90560:90561].reshape(1, 1)
    return _mlp(p2, jnp.asarray(_EXPAND), w1, b1, w2, b2, w3, b3, g)


# speedup vs baseline: 1.2299x; 1.0315x over previous
import functools

import numpy as np
import jax
import jax.numpy as jnp
from jax import lax
from jax.experimental import pallas as pl
from jax.experimental.pallas import tpu as pltpu
from jax.experimental.pallas import tpu_sc as plsc

NC, NS, L = 2, 16, 16
NW = NC * NS
BINS = 64
NIMG = 16
NCH = NIMG * 3
W = 512
ROWS_PER_CH = 512
TASKS_PER_CH = 2
NTASKS = NCH * TASKS_PER_CH
ROWS_PER_TASK = ROWS_PER_CH // TASKS_PER_CH
TASKS_PER_W = NTASKS // NW
ROWS = 64
CHUNKS_PER_TASK = ROWS_PER_TASK // ROWS
VPR = W // 16
NPH = 8
PHW = BINS * L
TASK_WORDS = PHW


def _sc_hist_body(img_hbm, out_hbm, buf0, buf1, acc, sem0, sem1):
    c = lax.axis_index("c")
    s = lax.axis_index("s")
    w = c * NS + s
    lane = lax.iota(jnp.int32, 16)
    lane_u = [lane + u * PHW for u in range(NPH)]
    zero = jnp.zeros((16,), jnp.float32)
    one = jnp.ones((16,), jnp.float32)

    bufs = (buf0, buf1)
    sems = (sem0, sem1)
    nchunks = TASKS_PER_W * CHUNKS_PER_TASK

    def start(k):
        ti, ci = divmod(k, CHUNKS_PER_TASK)
        t = w + NW * ti
        ch = t // TASKS_PER_CH
        row0 = (t % TASKS_PER_CH) * ROWS_PER_TASK + ci * ROWS
        cp = pltpu.make_async_copy(
            img_hbm.at[ch, pl.ds(row0, ROWS), :], bufs[k % 2], sems[k % 2])
        cp.start()
        return cp

    pending = [None, None]
    pending[0] = start(0)
    for k in range(nchunks):
        if k + 1 < nchunks:
            pending[(k + 1) % 2] = start(k + 1)

        if k % CHUNKS_PER_TASK == 0:
            @plsc.parallel_loop(0, NPH * PHW // 16, unroll=4)
            def _(i):
                acc[pl.ds(i * 16, 16)] = zero

        pending[k % 2].wait()
        buf = bufs[k % 2]

        @plsc.parallel_loop(0, ROWS * 2, step=1, unroll=2)
        def _(h, buf=buf):
            r = h >> 1
            c0 = (h & 1) * (W // 2)
            for u in range(VPR // 2):
                x = buf[r, pl.ds(c0 + u * 16, 16)]
                y = jnp.maximum(x * 64.0 + 524287.96875, 524288.0)
                b16 = plsc.bitcast(y, jnp.int32) & 0x3F0
                plsc.addupdate_scatter(acc, [lane_u[u % NPH] | b16], one)

        if k % CHUNKS_PER_TASK == CHUNKS_PER_TASK - 1:
            @plsc.parallel_loop(0, PHW // 16, unroll=4)
            def _(i):
                tot = acc[pl.ds(i * 16, 16)]
                for u in range(1, NPH):
                    tot = tot + acc[pl.ds(u * PHW + i * 16, 16)]
                acc[pl.ds(i * 16, 16)] = tot

            ti = k // CHUNKS_PER_TASK
            t = w + NW * ti
            pltpu.sync_copy(acc.at[pl.ds(0, TASK_WORDS)], out_hbm.at[t])


_sc_hist = functools.partial(
    pl.kernel,
    out_type=jax.ShapeDtypeStruct((NTASKS, TASK_WORDS), jnp.float32),
    mesh=plsc.VectorSubcoreMesh(
        core_axis_name="c", subcore_axis_name="s",
        num_cores=NC, num_subcores=NS),
    scratch_types=[
        pltpu.VMEM((ROWS, W), jnp.float32),
        pltpu.VMEM((ROWS, W), jnp.float32),
        pltpu.VMEM((NPH * PHW,), jnp.float32),
        pltpu.SemaphoreType.DMA,
        pltpu.SemaphoreType.DMA,
    ],
    compiler_params=pltpu.CompilerParams(
        needs_layout_passes=False, use_tc_tiling_on_sc=True),
)(_sc_hist_body)

_J = np.arange(3 * TASKS_PER_CH * PHW)
_EXPAND = np.zeros((3 * TASKS_PER_CH * PHW, 3 * BINS), np.float32)
_EXPAND[_J, (_J // (TASKS_PER_CH * PHW)) * BINS + (_J % PHW) // L] = 1.0


def _mlp_body(p_ref, m_ref, w1, b1, w2, b2, w3, b3, g, o_ref):
    q = p_ref[...]
    hist = jnp.dot(q, m_ref[...], preferred_element_type=jnp.float32)
    f = jnp.dot(hist, w1[...], preferred_element_type=jnp.float32) + b1[...]
    f = jnp.maximum(f, 0.0)
    f = jnp.dot(f, w2[...], preferred_element_type=jnp.float32) + b2[...]
    f = jnp.maximum(f, 0.0)
    f = jnp.dot(f, w3[...], preferred_element_type=jnp.float32) + b3[...]
    o_ref[...] = jax.nn.sigmoid(g[0, 0] + f)


_mlp = pl.pallas_call(
    _mlp_body,
    out_shape=jax.ShapeDtypeStruct((NIMG, BINS), jnp.float32),
)


def kernel(img, params):
    img3 = img.reshape(NCH, ROWS_PER_CH, W)
    partial = _sc_hist(img3)
    p2 = partial.reshape(NIMG, 3 * TASKS_PER_CH * PHW)
    w1 = params[0:49152].reshape(192, 256)
    b1 = params[49152:49408].reshape(1, 256)
    w2 = params[49408:82176].reshape(256, 128)
    b2 = params[82176:82304].reshape(1, 128)
    w3 = params[82304:90496].reshape(128, 64)
    b3 = params[90496:90560].reshape(1, 64)
    g = params[90560:90561].reshape(1, 1)
    return _mlp(p2, jnp.asarray(_EXPAND), w1, b1, w2, b2, w3, b3, g)
